# baseline (speedup 1.0000x reference)
import functools

import jax
import jax.numpy as jnp
from jax import lax
from jax.experimental import pallas as pl
from jax.experimental.pallas import tpu as pltpu
from jax.experimental.pallas import tpu_sc as plsc

N = 10000
E = 320000
D = 128
H = 128

_EDGE_BLOCK = 32768

_NUM_WORKERS = 32
_EPW = E // _NUM_WORKERS
_LANES = 16


def _node_proj_body(x_ref, w_ref, b_ref, wp_ref, o1_ref, o2_ref):
    x = x_ref[...].astype(jnp.bfloat16)
    w = w_ref[...].astype(jnp.bfloat16)
    n = jnp.dot(x, w, preferred_element_type=jnp.float32)
    n = jnp.maximum(n + b_ref[...], 0.0)
    nt = n.T
    o1_ref[...] = jnp.sum(nt * wp_ref[0:H], axis=0)
    o2_ref[...] = jnp.sum(nt * wp_ref[H:2 * H], axis=0)


def _edge_proj_body(x_ref, ei_ref, w_ref, b_ref, wp_ref, bp_ref,
                    o_ref, comb_ref):
    x = x_ref[...].astype(jnp.bfloat16)
    w = w_ref[...].astype(jnp.bfloat16)
    e = jnp.dot(x, w, preferred_element_type=jnp.float32)
    e = jnp.maximum(e + b_ref[...], 0.0)
    o_ref[...] = jnp.sum(e.T * wp_ref[2 * H:3 * H], axis=0) + bp_ref[...]
    comb_ref[...] = ei_ref[0] | (ei_ref[1] << 16)


def _make_combine_body(epw):
    unroll = 5
    assert epw % (_LANES * unroll) == 0 and epw % 8 == 0

    def _combine_body(p1_hbm, p2_hbm, comb_hbm, pe_hbm, out_hbm,
                      tab1_v, tab2_v, comb_v, pe_v, out_v, sem):
        wid = lax.axis_index("s") * 2 + lax.axis_index("c")
        base = wid * epw
        sl = pl.ds(base, epw)
        copies = [
            pltpu.async_copy(p1_hbm, tab1_v, sem),
            pltpu.async_copy(p2_hbm, tab2_v, sem),
            pltpu.async_copy(comb_hbm.at[sl], comb_v, sem),
            pltpu.async_copy(pe_hbm.at[sl], pe_v, sem),
        ]
        for c in copies:
            c.wait()

        def body(i, carry):
            for j in range(unroll):
                o = (i * unroll + j) * _LANES
                c = comb_v[pl.ds(o, _LANES)]
                s = c & 0xFFFF
                d = lax.shift_right_logical(c, 16)
                a = plsc.load_gather(tab1_v, [s])
                b = plsc.load_gather(tab2_v, [d])
                out_v[pl.ds(o, _LANES)] = a + b + pe_v[pl.ds(o, _LANES)]
            return carry

        lax.fori_loop(0, epw // (_LANES * unroll), body, 0)
        pltpu.sync_copy(out_v, out_hbm.at[sl])

    return _combine_body


def kernel(node_features, edge_features, edge_index, W_node, b_node,
           W_edge, b_edge, W_pred, b_pred):
    _NODE_BLOCK = 5120
    p1, p2 = pl.pallas_call(
        _node_proj_body,
        grid=(pl.cdiv(N, _NODE_BLOCK),),
        in_specs=[
            pl.BlockSpec((_NODE_BLOCK, D), lambda i: (i, 0)),
            pl.BlockSpec((D, H), lambda i: (0, 0)),
            pl.BlockSpec((1, H), lambda i: (0, 0)),
            pl.BlockSpec((3 * H, 1), lambda i: (0, 0)),
        ],
        out_specs=(pl.BlockSpec((_NODE_BLOCK,), lambda i: (i,)),
                   pl.BlockSpec((_NODE_BLOCK,), lambda i: (i,))),
        out_shape=(jax.ShapeDtypeStruct((N,), jnp.float32),
                   jax.ShapeDtypeStruct((N,), jnp.float32)),
    )(node_features, W_node, b_node.reshape(1, H), W_pred)

    def edge_call(nblocks, length, block_off):
        return pl.pallas_call(
            _edge_proj_body,
            grid=(nblocks,),
            in_specs=[
                pl.BlockSpec((_EDGE_BLOCK, D), lambda i, o=block_off: (i + o, 0)),
                pl.BlockSpec((2, _EDGE_BLOCK), lambda i, o=block_off: (0, i + o)),
                pl.BlockSpec((D, H), lambda i: (0, 0)),
                pl.BlockSpec((1, H), lambda i: (0, 0)),
                pl.BlockSpec((3 * H, 1), lambda i: (0, 0)),
                pl.BlockSpec((1,), lambda i: (0,)),
            ],
            out_specs=(pl.BlockSpec((_EDGE_BLOCK,), lambda i: (i,)),
                       pl.BlockSpec((_EDGE_BLOCK,), lambda i: (i,))),
            out_shape=(jax.ShapeDtypeStruct((length,), jnp.float32),
                       jax.ShapeDtypeStruct((length,), jnp.int32)),
        )(edge_features, edge_index, W_edge, b_edge.reshape(1, H), W_pred,
          b_pred)

    def combine_call(length):
        epw = length // _NUM_WORKERS
        return functools.partial(
            pl.kernel,
            out_type=jax.ShapeDtypeStruct((length,), jnp.float32),
            mesh=plsc.VectorSubcoreMesh(core_axis_name="c",
                                        subcore_axis_name="s"),
            compiler_params=pltpu.CompilerParams(needs_layout_passes=False),
            scratch_types=[
                pltpu.VMEM((N,), jnp.float32),
                pltpu.VMEM((N,), jnp.float32),
                pltpu.VMEM((epw,), jnp.int32),
                pltpu.VMEM((epw,), jnp.float32),
                pltpu.VMEM((epw,), jnp.float32),
                pltpu.SemaphoreType.DMA,
            ],
        )(_make_combine_body(epw))

    h1_blocks = 5
    h1 = h1_blocks * _EDGE_BLOCK
    h2 = E - h1
    pe1, comb1 = edge_call(h1_blocks, h1, 0)
    pe2, comb2 = edge_call(pl.cdiv(h2, _EDGE_BLOCK), h2, h1_blocks)
    out1 = combine_call(h1)(p1, p2, comb1, pe1)
    out2 = combine_call(h2)(p1, p2, comb2, pe2)
    return jnp.concatenate([out1, out2]).reshape(E, 1)


# speedup vs baseline: 1.0787x; 1.0787x over previous
import functools

import jax
import jax.numpy as jnp
from jax import lax
from jax.experimental import pallas as pl
from jax.experimental.pallas import tpu as pltpu
from jax.experimental.pallas import tpu_sc as plsc

N = 10000
E = 320000
D = 128
H = 128

_EDGE_BLOCK = 32768

_NUM_WORKERS = 32
_EPW = E // _NUM_WORKERS
_LANES = 16


def _node_proj_body(x_ref, w_ref, b_ref, wp_ref, o1_ref, o2_ref):
    x = x_ref[...].astype(jnp.bfloat16)
    w = w_ref[...].astype(jnp.bfloat16)
    n = jnp.dot(x, w, preferred_element_type=jnp.float32)
    n = jnp.maximum(n + b_ref[...], 0.0)
    nt = n.T
    o1_ref[...] = jnp.sum(nt * wp_ref[0:H], axis=0)
    o2_ref[...] = jnp.sum(nt * wp_ref[H:2 * H], axis=0)


def _edge_proj_body(x_ref, w_ref, b_ref, wp_ref, bp_ref, o_ref):
    x = x_ref[...].astype(jnp.bfloat16)
    w = w_ref[...].astype(jnp.bfloat16)
    e = jnp.dot(x, w, preferred_element_type=jnp.float32)
    e = jnp.maximum(e + b_ref[...], 0.0)
    o_ref[...] = jnp.sum(e.T * wp_ref[2 * H:3 * H], axis=0) + bp_ref[...]


def _repack_body(ei_ref, comb_ref):
    comb_ref[...] = ei_ref[0] | (ei_ref[1] << 16)


def _gather_body(p1_hbm, p2_hbm, comb_hbm, out_hbm,
                 tab1_v, tab2_v, comb_v, out_v, sem):
    wid = lax.axis_index("s") * 2 + lax.axis_index("c")
    base = wid * _EPW
    sl = pl.ds(base, _EPW)
    copies = [
        pltpu.async_copy(p1_hbm, tab1_v, sem),
        pltpu.async_copy(p2_hbm, tab2_v, sem),
        pltpu.async_copy(comb_hbm.at[sl], comb_v, sem),
    ]
    for c in copies:
        c.wait()

    _UNROLL = 5

    def body(i, carry):
        for j in range(_UNROLL):
            o = (i * _UNROLL + j) * _LANES
            c = comb_v[pl.ds(o, _LANES)]
            s = c & 0xFFFF
            d = lax.shift_right_logical(c, 16)
            a = plsc.load_gather(tab1_v, [s])
            b = plsc.load_gather(tab2_v, [d])
            out_v[pl.ds(o, _LANES)] = a + b
        return carry

    lax.fori_loop(0, _EPW // (_LANES * _UNROLL), body, 0)
    pltpu.sync_copy(out_v, out_hbm.at[sl])


def _add_body(g_hbm, pe_hbm, out_hbm, g_v, pe_v, out_v, sem):
    wid = lax.axis_index("s") * 2 + lax.axis_index("c")
    base = wid * _EPW
    sl = pl.ds(base, _EPW)
    copies = [
        pltpu.async_copy(g_hbm.at[sl], g_v, sem),
        pltpu.async_copy(pe_hbm.at[sl], pe_v, sem),
    ]
    for c in copies:
        c.wait()

    _UNROLL = 5

    def body(i, carry):
        for j in range(_UNROLL):
            o = (i * _UNROLL + j) * _LANES
            out_v[pl.ds(o, _LANES)] = g_v[pl.ds(o, _LANES)] + pe_v[pl.ds(o, _LANES)]
        return carry

    lax.fori_loop(0, _EPW // (_LANES * _UNROLL), body, 0)
    pltpu.sync_copy(out_v, out_hbm.at[sl])


def kernel(node_features, edge_features, edge_index, W_node, b_node,
           W_edge, b_edge, W_pred, b_pred):
    _NODE_BLOCK = 5120
    p1, p2 = pl.pallas_call(
        _node_proj_body,
        grid=(pl.cdiv(N, _NODE_BLOCK),),
        in_specs=[
            pl.BlockSpec((_NODE_BLOCK, D), lambda i: (i, 0)),
            pl.BlockSpec((D, H), lambda i: (0, 0)),
            pl.BlockSpec((1, H), lambda i: (0, 0)),
            pl.BlockSpec((3 * H, 1), lambda i: (0, 0)),
        ],
        out_specs=(pl.BlockSpec((_NODE_BLOCK,), lambda i: (i,)),
                   pl.BlockSpec((_NODE_BLOCK,), lambda i: (i,))),
        out_shape=(jax.ShapeDtypeStruct((N,), jnp.float32),
                   jax.ShapeDtypeStruct((N,), jnp.float32)),
    )(node_features, W_node, b_node.reshape(1, H), W_pred)

    comb = pl.pallas_call(
        _repack_body,
        grid=(pl.cdiv(E, _EDGE_BLOCK),),
        in_specs=[pl.BlockSpec((2, _EDGE_BLOCK), lambda i: (0, i))],
        out_specs=pl.BlockSpec((_EDGE_BLOCK,), lambda i: (i,)),
        out_shape=jax.ShapeDtypeStruct((E,), jnp.int32),
    )(edge_index)

    gsum = functools.partial(
        pl.kernel,
        out_type=jax.ShapeDtypeStruct((E,), jnp.float32),
        mesh=plsc.VectorSubcoreMesh(core_axis_name="c", subcore_axis_name="s"),
        compiler_params=pltpu.CompilerParams(needs_layout_passes=False),
        scratch_types=[
            pltpu.VMEM((N,), jnp.float32),
            pltpu.VMEM((N,), jnp.float32),
            pltpu.VMEM((_EPW,), jnp.int32),
            pltpu.VMEM((_EPW,), jnp.float32),
            pltpu.SemaphoreType.DMA,
        ],
    )(_gather_body)(p1, p2, comb)

    pe = pl.pallas_call(
        _edge_proj_body,
        grid=(pl.cdiv(E, _EDGE_BLOCK),),
        in_specs=[
            pl.BlockSpec((_EDGE_BLOCK, D), lambda i: (i, 0)),
            pl.BlockSpec((D, H), lambda i: (0, 0)),
            pl.BlockSpec((1, H), lambda i: (0, 0)),
            pl.BlockSpec((3 * H, 1), lambda i: (0, 0)),
            pl.BlockSpec((1,), lambda i: (0,)),
        ],
        out_specs=pl.BlockSpec((_EDGE_BLOCK,), lambda i: (i,)),
        out_shape=jax.ShapeDtypeStruct((E,), jnp.float32),
    )(edge_features, W_edge, b_edge.reshape(1, H), W_pred, b_pred)

    out = functools.partial(
        pl.kernel,
        out_type=jax.ShapeDtypeStruct((E,), jnp.float32),
        mesh=plsc.VectorSubcoreMesh(core_axis_name="c", subcore_axis_name="s"),
        compiler_params=pltpu.CompilerParams(needs_layout_passes=False),
        scratch_types=[
            pltpu.VMEM((_EPW,), jnp.float32),
            pltpu.VMEM((_EPW,), jnp.float32),
            pltpu.VMEM((_EPW,), jnp.float32),
            pltpu.SemaphoreType.DMA,
        ],
    )(_add_body)(gsum, pe)
    return out.reshape(E, 1)
